# Initial kernel scaffold; baseline (speedup 1.0000x reference)
import jax
import jax.numpy as jnp
from jax.experimental import pallas as pl


def kernel(x, edge_index_r0, edge_index_r1, edge_index_r2, W_r0, W_r1, W_r2, b_r0, b_r1, b_r2, W_att, v_att):
    raise NotImplementedError("write your pallas kernel here")


# speedup vs baseline: 2.3143x; 2.3143x over previous
import functools

import jax
import jax.numpy as jnp
from jax import lax
from jax.experimental import pallas as pl
from jax.experimental.pallas import tpu as pltpu
from jax.experimental.pallas import tpu_sc as plsc

N = 10000
F = 128
HID = 256
ATT = 64
E = 320000

NPAD = 10240
EPAD = 327680
EROWS = EPAD // 128
NW = 32
ROWS_PER_W = EPAD // 128 // NW
OUTER = 5
INNER = ROWS_PER_W // OUTER
TSLICE = NPAD // 16

_mesh = plsc.VectorSubcoreMesh(
    core_axis_name="c", subcore_axis_name="s", num_cores=2, num_subcores=16)


@functools.partial(
    pl.kernel,
    out_type=jax.ShapeDtypeStruct((2, 6 * NPAD), jnp.float32),
    mesh=_mesh,
    scratch_types=[
        pltpu.VMEM((INNER, 128), jnp.int32),
        pltpu.VMEM((128,), jnp.float32),
        pltpu.VMEM((TSLICE,), jnp.float32),
        [pltpu.VMEM_SHARED((NPAD,), jnp.float32) for _ in range(6)],
    ],
)
def _hist_kernel(e0, e1, e2, e3, e4, e5, out, idx_v, ones_v, zero_v, hists):
    c = lax.axis_index("c")
    s = lax.axis_index("s")
    w = s * 2 + c

    ones16 = jnp.ones((16,), jnp.float32)
    zero16 = jnp.zeros((16,), jnp.float32)

    def init_ones(i, _):
        ones_v[pl.ds(i * 16, 16)] = ones16
        return _

    lax.fori_loop(0, 128 // 16, init_ones, None)

    def init_zero(i, _):
        zero_v[pl.ds(i * 16, 16)] = zero16
        return _

    lax.fori_loop(0, TSLICE // 16, init_zero, None)

    for a in range(6):
        pltpu.sync_copy(zero_v, hists[a].at[pl.ds(s * TSLICE, TSLICE)])
    plsc.subcore_barrier()

    for a, ehbm in enumerate((e0, e1, e2, e3, e4, e5)):
        h_ref = hists[a]

        def outer(cc, _, ehbm=ehbm, h_ref=h_ref):
            base = w * ROWS_PER_W + cc * INNER
            pltpu.sync_copy(ehbm.at[pl.ds(base, INNER)], idx_v)
            for j in range(INNER):
                pltpu.sync_copy(ones_v, h_ref.at[idx_v.at[j]], add=True)
            return _

        lax.fori_loop(0, OUTER, outer, None)

    plsc.subcore_barrier()
    for a in range(6):
        pltpu.sync_copy(
            hists[a].at[pl.ds(s * TSLICE, TSLICE)],
            out.at[c, pl.ds(a * NPAD + s * TSLICE, TSLICE)])


@functools.partial(
    pl.kernel,
    out_type=[jax.ShapeDtypeStruct((2, NPAD, F), jnp.float32)
              for _ in range(3)],
    mesh=_mesh,
    scratch_types=[
        pltpu.VMEM((INNER, 128), jnp.int32),
        pltpu.VMEM((INNER, 128), jnp.int32),
        pltpu.VMEM((128, F), jnp.float32),
        pltpu.VMEM((128, F), jnp.float32),
        pltpu.VMEM_SHARED((NPAD, F), jnp.float32),
        pltpu.SemaphoreType.DMA,
    ],
)
def _segsum_kernel(xn0, xn1, xn2, s0, d0, s1, d1, s2, d2,
                   o0, o1, o2, si_v, di_v, rows_v, zero_v, agg, sem):
    c = lax.axis_index("c")
    s = lax.axis_index("s")
    w = s * 2 + c

    z16 = jnp.zeros((16,), jnp.float32)

    def zrow(r, _):
        for q in range(F // 16):
            zero_v[r, pl.ds(q * 16, 16)] = z16
        return _

    lax.fori_loop(0, 128, zrow, None)

    xns = (xn0, xn1, xn2)
    srcs = (s0, s1, s2)
    dsts = (d0, d1, d2)
    outs = (o0, o1, o2)

    for r in range(3):
        for k in range(TSLICE // 128):
            pltpu.sync_copy(zero_v, agg.at[pl.ds(s * TSLICE + k * 128, 128)])
        plsc.subcore_barrier()

        def outer(cc, _, r=r):
            base = w * ROWS_PER_W + cc * INNER
            pltpu.sync_copy(srcs[r].at[pl.ds(base, INNER)], si_v)
            pltpu.sync_copy(dsts[r].at[pl.ds(base, INNER)], di_v)
            for j in range(INNER):
                pltpu.async_copy(xns[r].at[si_v.at[j]], rows_v, sem).wait()
                pltpu.sync_copy(rows_v, agg.at[di_v.at[j]], add=True)
            return _

        lax.fori_loop(0, OUTER, outer, None)
        plsc.subcore_barrier()
        pltpu.sync_copy(agg.at[pl.ds(s * TSLICE, TSLICE)],
                        outs[r].at[c, pl.ds(s * TSLICE, TSLICE)])
        if r < 2:
            plsc.subcore_barrier()


PB = 2048


def _pre_body(x_ref, hist_ref, xn0_ref, xn1_ref, xn2_ref, sin_ref):
    h = hist_ref[...]
    deg = h[0] + h[1]
    sc = lax.rsqrt(jnp.maximum(deg, 1.0))
    xv = x_ref[...]
    xn0_ref[...] = xv * sc[0][:, None]
    xn1_ref[...] = xv * sc[1][:, None]
    xn2_ref[...] = xv * sc[2][:, None]
    sin_ref[...] = sc[3:6]


def _tc_pre(x_pad, hist):
    grid = (NPAD // PB,)
    return pl.pallas_call(
        _pre_body,
        grid=grid,
        in_specs=[
            pl.BlockSpec((PB, F), lambda i: (i, 0)),
            pl.BlockSpec((2, 6, PB), lambda i: (0, 0, i)),
        ],
        out_specs=[
            pl.BlockSpec((PB, F), lambda i: (i, 0)),
            pl.BlockSpec((PB, F), lambda i: (i, 0)),
            pl.BlockSpec((PB, F), lambda i: (i, 0)),
            pl.BlockSpec((3, PB), lambda i: (0, i)),
        ],
        out_shape=[jax.ShapeDtypeStruct((NPAD, F), jnp.float32)] * 3
        + [jax.ShapeDtypeStruct((3, NPAD), jnp.float32)],
    )(x_pad, hist)


PB2 = 1024


def _post_body(a0_ref, a1_ref, a2_ref, sin_ref, w_ref, b_ref, wa_ref, va_ref,
               out_ref):
    sin = sin_ref[...]
    hs = []
    for r, ar in enumerate((a0_ref, a1_ref, a2_ref)):
        av = ar[...]
        asum = (av[0] + av[1]) * sin[r][:, None]
        hs.append(jnp.dot(asum, w_ref[r], preferred_element_type=jnp.float32)
                  + b_ref[r][None, :])
    h = jnp.stack(hs)
    t = jnp.tanh(jnp.dot(h, wa_ref[...],
                         preferred_element_type=jnp.float32))
    e = jnp.sum(t * va_ref[0][None, None, :], axis=-1)
    m = jnp.max(e, axis=0)
    ew = jnp.exp(e - m[None, :])
    alpha = ew / jnp.sum(ew, axis=0)[None, :]
    out_ref[...] = (alpha[0][:, None] * h[0] + alpha[1][:, None] * h[1]
                    + alpha[2][:, None] * h[2])


def _tc_post(a0, a1, a2, sin, W, b, Wa, va):
    grid = (NPAD // PB2,)
    return pl.pallas_call(
        _post_body,
        grid=grid,
        in_specs=[
            pl.BlockSpec((2, PB2, F), lambda i: (0, i, 0)),
            pl.BlockSpec((2, PB2, F), lambda i: (0, i, 0)),
            pl.BlockSpec((2, PB2, F), lambda i: (0, i, 0)),
            pl.BlockSpec((3, PB2), lambda i: (0, i)),
            pl.BlockSpec((3, F, HID), lambda i: (0, 0, 0)),
            pl.BlockSpec((3, HID), lambda i: (0, 0)),
            pl.BlockSpec((HID, ATT), lambda i: (0, 0)),
            pl.BlockSpec((1, ATT), lambda i: (0, 0)),
        ],
        out_specs=pl.BlockSpec((PB2, HID), lambda i: (i, 0)),
        out_shape=jax.ShapeDtypeStruct((NPAD, HID), jnp.float32),
    )(a0, a1, a2, sin, W, b, Wa, va)


def kernel(x, edge_index_r0, edge_index_r1, edge_index_r2,
           W_r0, W_r1, W_r2, b_r0, b_r1, b_r2, W_att, v_att):
    pad = jnp.full((EPAD - E,), NPAD - 1, jnp.int32)
    idx = []
    for ei in (edge_index_r0, edge_index_r1, edge_index_r2):
        idx.append(jnp.concatenate([ei[0], pad]).reshape(EROWS, 128))
        idx.append(jnp.concatenate([ei[1], pad]).reshape(EROWS, 128))

    hist = _hist_kernel(idx[0], idx[2], idx[4], idx[1], idx[3], idx[5])
    hist = hist.reshape(2, 6, NPAD)

    x_pad = jnp.zeros((NPAD, F), jnp.float32).at[:N].set(x)
    xn0, xn1, xn2, sin = _tc_pre(x_pad, hist)

    a0, a1, a2 = _segsum_kernel(
        xn0, xn1, xn2, idx[0], idx[1], idx[2], idx[3], idx[4], idx[5])

    W = jnp.stack([W_r0, W_r1, W_r2])
    b = jnp.stack([b_r0, b_r1, b_r2])
    out = _tc_post(a0, a1, a2, sin, W, b, W_att, v_att.reshape(1, ATT))
    return out[:N]


# speedup vs baseline: 2.5676x; 1.1094x over previous
import functools

import jax
import jax.numpy as jnp
from jax import lax
from jax.experimental import pallas as pl
from jax.experimental.pallas import tpu as pltpu
from jax.experimental.pallas import tpu_sc as plsc

N = 10000
F = 128
HID = 256
ATT = 64
E = 320000

NPAD = 10240
EPAD = 327680
EW = 64
EROWS = EPAD // EW
NW = 32
ROWS_PER_W = EPAD // EW // NW
OUTER = 5
INNER = ROWS_PER_W // OUTER
TSLICE = NPAD // 16

_mesh = plsc.VectorSubcoreMesh(
    core_axis_name="c", subcore_axis_name="s", num_cores=2, num_subcores=16)


@functools.partial(
    pl.kernel,
    out_type=jax.ShapeDtypeStruct((2, 6 * NPAD), jnp.float32),
    mesh=_mesh,
    scratch_types=[
        pltpu.VMEM((INNER, EW), jnp.int32),
        pltpu.VMEM((EW,), jnp.float32),
        pltpu.VMEM((TSLICE,), jnp.float32),
        [pltpu.VMEM_SHARED((NPAD,), jnp.float32) for _ in range(6)],
    ],
)
def _hist_kernel(e0, e1, e2, e3, e4, e5, out, idx_v, ones_v, zero_v, hists):
    c = lax.axis_index("c")
    s = lax.axis_index("s")
    w = s * 2 + c

    ones16 = jnp.ones((16,), jnp.float32)
    zero16 = jnp.zeros((16,), jnp.float32)

    def init_ones(i, _):
        ones_v[pl.ds(i * 16, 16)] = ones16
        return _

    lax.fori_loop(0, EW // 16, init_ones, None)

    def init_zero(i, _):
        zero_v[pl.ds(i * 16, 16)] = zero16
        return _

    lax.fori_loop(0, TSLICE // 16, init_zero, None)

    for a in range(6):
        pltpu.sync_copy(zero_v, hists[a].at[pl.ds(s * TSLICE, TSLICE)])
    plsc.subcore_barrier()

    for a, ehbm in enumerate((e0, e1, e2, e3, e4, e5)):
        h_ref = hists[a]

        def outer(cc, _, ehbm=ehbm, h_ref=h_ref):
            base = w * ROWS_PER_W + cc * INNER
            pltpu.sync_copy(ehbm.at[pl.ds(base, INNER)], idx_v)
            for j in range(INNER):
                pltpu.sync_copy(ones_v, h_ref.at[idx_v.at[j]], add=True)
            return _

        lax.fori_loop(0, OUTER, outer, None)

    plsc.subcore_barrier()
    for a in range(6):
        pltpu.sync_copy(
            hists[a].at[pl.ds(s * TSLICE, TSLICE)],
            out.at[c, pl.ds(a * NPAD + s * TSLICE, TSLICE)])


@functools.partial(
    pl.kernel,
    out_type=[jax.ShapeDtypeStruct((2, NPAD, F), jnp.float32)
              for _ in range(3)],
    mesh=_mesh,
    scratch_types=[
        pltpu.VMEM((INNER, EW), jnp.int32),
        pltpu.VMEM((INNER, EW), jnp.int32),
        [pltpu.VMEM((EW, F), jnp.float32) for _ in range(4)],
        pltpu.VMEM((16, F), jnp.float32),
        pltpu.VMEM_SHARED((NPAD, F), jnp.float32),
        [pltpu.SemaphoreType.DMA for _ in range(4)],
        [pltpu.SemaphoreType.DMA for _ in range(4)],
    ],
)
def _segsum_kernel(xn0, xn1, xn2, s0, d0, s1, d1, s2, d2,
                   o0, o1, o2, si_v, di_v, bufs, zero_v, agg, gsems, ssems):
    c = lax.axis_index("c")
    s = lax.axis_index("s")
    w = s * 2 + c
    nbuf = len(bufs)

    z16 = jnp.zeros((16,), jnp.float32)

    def zrow(r, _):
        for q in range(F // 16):
            zero_v[r, pl.ds(q * 16, 16)] = z16
        return _

    lax.fori_loop(0, 16, zrow, None)

    xns = (xn0, xn1, xn2)
    srcs = (s0, s1, s2)
    dsts = (d0, d1, d2)
    outs = (o0, o1, o2)

    for r in range(3):
        def zslice(k, _):
            pltpu.sync_copy(zero_v, agg.at[pl.ds(s * TSLICE + k * 16, 16)])
            return _

        lax.fori_loop(0, TSLICE // 16, zslice, None)
        plsc.subcore_barrier()

        def outer(cc, _, r=r):
            base = w * ROWS_PER_W + cc * INNER
            pltpu.sync_copy(srcs[r].at[pl.ds(base, INNER)], si_v)
            pltpu.sync_copy(dsts[r].at[pl.ds(base, INNER)], di_v)
            gd = [None] * INNER
            sd = [None] * INNER
            for j in range(nbuf):
                b = j % nbuf
                gd[j] = pltpu.async_copy(
                    xns[r].at[si_v.at[j]], bufs[b], gsems[b])
            for j in range(INNER):
                b = j % nbuf
                gd[j].wait()
                sd[j] = pltpu.async_copy(
                    bufs[b], agg.at[di_v.at[j]], ssems[b], add=True)
                jf = j - 2
                if jf >= 0:
                    sd[jf].wait()
                    nxt = jf + nbuf
                    if nxt < INNER:
                        nb = nxt % nbuf
                        gd[nxt] = pltpu.async_copy(
                            xns[r].at[si_v.at[nxt]], bufs[nb], gsems[nb])
            sd[INNER - 2].wait()
            sd[INNER - 1].wait()
            return _

        lax.fori_loop(0, OUTER, outer, None)
        plsc.subcore_barrier()
        pltpu.sync_copy(agg.at[pl.ds(s * TSLICE, TSLICE)],
                        outs[r].at[c, pl.ds(s * TSLICE, TSLICE)])
        if r < 2:
            plsc.subcore_barrier()


PB = 2048


def _pre_body(x_ref, hist_ref, xn0_ref, xn1_ref, xn2_ref, sin_ref):
    h = hist_ref[...]
    deg = h[0] + h[1]
    sc = lax.rsqrt(jnp.maximum(deg, 1.0))
    xv = x_ref[...]
    xn0_ref[...] = xv * sc[0][:, None]
    xn1_ref[...] = xv * sc[1][:, None]
    xn2_ref[...] = xv * sc[2][:, None]
    sin_ref[...] = sc[3:6]


def _tc_pre(x_pad, hist):
    grid = (NPAD // PB,)
    return pl.pallas_call(
        _pre_body,
        grid=grid,
        in_specs=[
            pl.BlockSpec((PB, F), lambda i: (i, 0)),
            pl.BlockSpec((2, 6, PB), lambda i: (0, 0, i)),
        ],
        out_specs=[
            pl.BlockSpec((PB, F), lambda i: (i, 0)),
            pl.BlockSpec((PB, F), lambda i: (i, 0)),
            pl.BlockSpec((PB, F), lambda i: (i, 0)),
            pl.BlockSpec((3, PB), lambda i: (0, i)),
        ],
        out_shape=[jax.ShapeDtypeStruct((NPAD, F), jnp.float32)] * 3
        + [jax.ShapeDtypeStruct((3, NPAD), jnp.float32)],
    )(x_pad, hist)


PB2 = 1024


def _post_body(a0_ref, a1_ref, a2_ref, sin_ref, w_ref, b_ref, wa_ref, va_ref,
               out_ref):
    sin = sin_ref[...]
    hs = []
    for r, ar in enumerate((a0_ref, a1_ref, a2_ref)):
        av = ar[...]
        asum = (av[0] + av[1]) * sin[r][:, None]
        hs.append(jnp.dot(asum, w_ref[r], preferred_element_type=jnp.float32)
                  + b_ref[r][None, :])
    h = jnp.stack(hs)
    t = jnp.tanh(jnp.dot(h, wa_ref[...],
                         preferred_element_type=jnp.float32))
    e = jnp.sum(t * va_ref[0][None, None, :], axis=-1)
    m = jnp.max(e, axis=0)
    ew = jnp.exp(e - m[None, :])
    alpha = ew / jnp.sum(ew, axis=0)[None, :]
    out_ref[...] = (alpha[0][:, None] * h[0] + alpha[1][:, None] * h[1]
                    + alpha[2][:, None] * h[2])


def _tc_post(a0, a1, a2, sin, W, b, Wa, va):
    grid = (NPAD // PB2,)
    return pl.pallas_call(
        _post_body,
        grid=grid,
        in_specs=[
            pl.BlockSpec((2, PB2, F), lambda i: (0, i, 0)),
            pl.BlockSpec((2, PB2, F), lambda i: (0, i, 0)),
            pl.BlockSpec((2, PB2, F), lambda i: (0, i, 0)),
            pl.BlockSpec((3, PB2), lambda i: (0, i)),
            pl.BlockSpec((3, F, HID), lambda i: (0, 0, 0)),
            pl.BlockSpec((3, HID), lambda i: (0, 0)),
            pl.BlockSpec((HID, ATT), lambda i: (0, 0)),
            pl.BlockSpec((1, ATT), lambda i: (0, 0)),
        ],
        out_specs=pl.BlockSpec((PB2, HID), lambda i: (i, 0)),
        out_shape=jax.ShapeDtypeStruct((NPAD, HID), jnp.float32),
    )(a0, a1, a2, sin, W, b, Wa, va)


def kernel(x, edge_index_r0, edge_index_r1, edge_index_r2,
           W_r0, W_r1, W_r2, b_r0, b_r1, b_r2, W_att, v_att):
    pad = jnp.full((EPAD - E,), NPAD - 1, jnp.int32)
    idx = []
    for ei in (edge_index_r0, edge_index_r1, edge_index_r2):
        idx.append(jnp.concatenate([ei[0], pad]).reshape(EROWS, EW))
        idx.append(jnp.concatenate([ei[1], pad]).reshape(EROWS, EW))

    hist = _hist_kernel(idx[0], idx[2], idx[4], idx[1], idx[3], idx[5])
    hist = hist.reshape(2, 6, NPAD)

    x_pad = jnp.zeros((NPAD, F), jnp.float32).at[:N].set(x)
    xn0, xn1, xn2, sin = _tc_pre(x_pad, hist)

    a0, a1, a2 = _segsum_kernel(
        xn0, xn1, xn2, idx[0], idx[1], idx[2], idx[3], idx[4], idx[5])

    W = jnp.stack([W_r0, W_r1, W_r2])
    b = jnp.stack([b_r0, b_r1, b_r2])
    out = _tc_post(a0, a1, a2, sin, W, b, W_att, v_att.reshape(1, ATT))
    return out[:N]


# speedup vs baseline: 2.7684x; 1.0782x over previous
import functools

import jax
import jax.numpy as jnp
from jax import lax
from jax.experimental import pallas as pl
from jax.experimental.pallas import tpu as pltpu
from jax.experimental.pallas import tpu_sc as plsc

N = 10000
F = 128
HID = 256
ATT = 64
E = 320000

NPAD = 10240
EPAD = 327680
EW = 64
EROWS = EPAD // EW
NW = 32
ROWS_PER_W = EPAD // EW // NW
OUTER = 5
INNER = ROWS_PER_W // OUTER
TSLICE = NPAD // 16

SEG_INNER = 16
ROWS_C0 = 3840
ROWS_C1 = EROWS - ROWS_C0
WROWS_C0 = ROWS_C0 // 16
WROWS_C1 = ROWS_C1 // 16
OUTER_C0 = WROWS_C0 // SEG_INNER
OUTER_C1 = WROWS_C1 // SEG_INNER

_mesh = plsc.VectorSubcoreMesh(
    core_axis_name="c", subcore_axis_name="s", num_cores=2, num_subcores=16)


@functools.partial(
    pl.kernel,
    out_type=jax.ShapeDtypeStruct((2, 6 * NPAD), jnp.float32),
    mesh=_mesh,
    scratch_types=[
        [pltpu.VMEM((INNER, EW), jnp.int32) for _ in range(2)],
        pltpu.VMEM((EW,), jnp.float32),
        pltpu.VMEM((TSLICE,), jnp.float32),
        [pltpu.VMEM_SHARED((NPAD,), jnp.float32) for _ in range(6)],
        pltpu.SemaphoreType.DMA,
    ],
)
def _hist_kernel(e0, e1, e2, e3, e4, e5, out, idx_v, ones_v, zero_v, hists,
                 ssem):
    c = lax.axis_index("c")
    s = lax.axis_index("s")
    w = s * 2 + c

    ones16 = jnp.ones((16,), jnp.float32)
    zero16 = jnp.zeros((16,), jnp.float32)

    def init_ones(i, _):
        ones_v[pl.ds(i * 16, 16)] = ones16
        return _

    lax.fori_loop(0, EW // 16, init_ones, None)

    def init_zero(i, _):
        zero_v[pl.ds(i * 16, 16)] = zero16
        return _

    lax.fori_loop(0, TSLICE // 16, init_zero, None)

    for a in range(6):
        pltpu.sync_copy(zero_v, hists[a].at[pl.ds(s * TSLICE, TSLICE)])
    plsc.subcore_barrier()

    wb = w * ROWS_PER_W
    for a, ehbm in enumerate((e0, e1, e2, e3, e4, e5)):
        h_ref = hists[a]

        def outer(cc, _, ehbm=ehbm, h_ref=h_ref):
            pltpu.sync_copy(ehbm.at[pl.ds(wb + cc * INNER, INNER)], idx_v[0])
            sds = [pltpu.async_copy(ones_v, h_ref.at[idx_v[0].at[j]], ssem,
                                    add=True)
                   for j in range(INNER)]
            for sdd in sds:
                sdd.wait()
            return _

        lax.fori_loop(0, OUTER, outer, None)

    plsc.subcore_barrier()
    for a in range(6):
        pltpu.sync_copy(
            hists[a].at[pl.ds(s * TSLICE, TSLICE)],
            out.at[c, pl.ds(a * NPAD + s * TSLICE, TSLICE)])


@functools.partial(
    pl.kernel,
    out_type=[jax.ShapeDtypeStruct((2, NPAD, F), jnp.float32)
              for _ in range(3)],
    mesh=_mesh,
    scratch_types=[
        pltpu.VMEM((SEG_INNER, EW), jnp.int32),
        pltpu.VMEM((SEG_INNER, EW), jnp.int32),
        [pltpu.VMEM((EW, F), jnp.float32) for _ in range(4)],
        pltpu.VMEM((16, F), jnp.float32),
        pltpu.VMEM_SHARED((NPAD, F), jnp.float32),
        [pltpu.SemaphoreType.DMA for _ in range(4)],
        [pltpu.SemaphoreType.DMA for _ in range(4)],
    ],
)
def _segsum_kernel(xn0, xn1, xn2, s0, d0, s1, d1, s2, d2,
                   o0, o1, o2, si_v, di_v, bufs, zero_v, agg, gsems, ssems):
    c = lax.axis_index("c")
    s = lax.axis_index("s")
    nbuf = len(bufs)

    wbase = jnp.where(c == 0, s * WROWS_C0, ROWS_C0 + s * WROWS_C1)
    n_outer = jnp.where(c == 0, OUTER_C0, OUTER_C1)

    z16 = jnp.zeros((16,), jnp.float32)

    def zrow(r, _):
        for q in range(F // 16):
            zero_v[r, pl.ds(q * 16, 16)] = z16
        return _

    lax.fori_loop(0, 16, zrow, None)

    xns = (xn0, xn1, xn2)
    srcs = (s0, s1, s2)
    dsts = (d0, d1, d2)
    outs = (o0, o1, o2)

    for r in range(3):
        def zslice(k, _):
            pltpu.sync_copy(zero_v, agg.at[pl.ds(s * TSLICE + k * 16, 16)])
            return _

        lax.fori_loop(0, TSLICE // 16, zslice, None)
        plsc.subcore_barrier()

        def outer(cc, _, r=r):
            base = wbase + cc * SEG_INNER
            pltpu.sync_copy(srcs[r].at[pl.ds(base, SEG_INNER)], si_v)
            pltpu.sync_copy(dsts[r].at[pl.ds(base, SEG_INNER)], di_v)
            gd = [None] * SEG_INNER
            sd = [None] * SEG_INNER
            for j in range(nbuf):
                b = j % nbuf
                gd[j] = pltpu.async_copy(
                    xns[r].at[si_v.at[j]], bufs[b], gsems[b])
            for j in range(SEG_INNER):
                b = j % nbuf
                gd[j].wait()
                sd[j] = pltpu.async_copy(
                    bufs[b], agg.at[di_v.at[j]], ssems[b], add=True)
                jf = j - 2
                if jf >= 0:
                    sd[jf].wait()
                    nxt = jf + nbuf
                    if nxt < SEG_INNER:
                        nb = nxt % nbuf
                        gd[nxt] = pltpu.async_copy(
                            xns[r].at[si_v.at[nxt]], bufs[nb], gsems[nb])
            sd[SEG_INNER - 2].wait()
            sd[SEG_INNER - 1].wait()
            return _

        lax.fori_loop(0, n_outer, outer, None)
        plsc.subcore_barrier()
        pltpu.sync_copy(agg.at[pl.ds(s * TSLICE, TSLICE)],
                        outs[r].at[c, pl.ds(s * TSLICE, TSLICE)])
        if r < 2:
            plsc.subcore_barrier()


PB = 2048


def _pre_body(x_ref, hist_ref, xn0_ref, xn1_ref, xn2_ref, sin_ref):
    h = hist_ref[...]
    deg = h[0] + h[1]
    sc = lax.rsqrt(jnp.maximum(deg, 1.0))
    xv = x_ref[...]
    xn0_ref[...] = xv * sc[0][:, None]
    xn1_ref[...] = xv * sc[1][:, None]
    xn2_ref[...] = xv * sc[2][:, None]
    sin_ref[...] = sc[3:6]


def _tc_pre(x_pad, hist):
    grid = (NPAD // PB,)
    return pl.pallas_call(
        _pre_body,
        grid=grid,
        in_specs=[
            pl.BlockSpec((PB, F), lambda i: (i, 0)),
            pl.BlockSpec((2, 6, PB), lambda i: (0, 0, i)),
        ],
        out_specs=[
            pl.BlockSpec((PB, F), lambda i: (i, 0)),
            pl.BlockSpec((PB, F), lambda i: (i, 0)),
            pl.BlockSpec((PB, F), lambda i: (i, 0)),
            pl.BlockSpec((3, PB), lambda i: (0, i)),
        ],
        out_shape=[jax.ShapeDtypeStruct((NPAD, F), jnp.float32)] * 3
        + [jax.ShapeDtypeStruct((3, NPAD), jnp.float32)],
    )(x_pad, hist)


PB2 = 1024


def _post_body(a0_ref, a1_ref, a2_ref, sin_ref, w_ref, b_ref, wa_ref, va_ref,
               out_ref):
    sin = sin_ref[...]
    hs = []
    for r, ar in enumerate((a0_ref, a1_ref, a2_ref)):
        av = ar[...]
        asum = (av[0] + av[1]) * sin[r][:, None]
        hs.append(jnp.dot(asum, w_ref[r], preferred_element_type=jnp.float32)
                  + b_ref[r][None, :])
    h = jnp.stack(hs)
    t = jnp.tanh(jnp.dot(h, wa_ref[...],
                         preferred_element_type=jnp.float32))
    e = jnp.sum(t * va_ref[0][None, None, :], axis=-1)
    m = jnp.max(e, axis=0)
    ew = jnp.exp(e - m[None, :])
    alpha = ew / jnp.sum(ew, axis=0)[None, :]
    out_ref[...] = (alpha[0][:, None] * h[0] + alpha[1][:, None] * h[1]
                    + alpha[2][:, None] * h[2])


def _tc_post(a0, a1, a2, sin, W, b, Wa, va):
    grid = (NPAD // PB2,)
    return pl.pallas_call(
        _post_body,
        grid=grid,
        in_specs=[
            pl.BlockSpec((2, PB2, F), lambda i: (0, i, 0)),
            pl.BlockSpec((2, PB2, F), lambda i: (0, i, 0)),
            pl.BlockSpec((2, PB2, F), lambda i: (0, i, 0)),
            pl.BlockSpec((3, PB2), lambda i: (0, i)),
            pl.BlockSpec((3, F, HID), lambda i: (0, 0, 0)),
            pl.BlockSpec((3, HID), lambda i: (0, 0)),
            pl.BlockSpec((HID, ATT), lambda i: (0, 0)),
            pl.BlockSpec((1, ATT), lambda i: (0, 0)),
        ],
        out_specs=pl.BlockSpec((PB2, HID), lambda i: (i, 0)),
        out_shape=jax.ShapeDtypeStruct((NPAD, HID), jnp.float32),
    )(a0, a1, a2, sin, W, b, Wa, va)


def kernel(x, edge_index_r0, edge_index_r1, edge_index_r2,
           W_r0, W_r1, W_r2, b_r0, b_r1, b_r2, W_att, v_att):
    pad = jnp.full((EPAD - E,), NPAD - 1, jnp.int32)
    idx = []
    for ei in (edge_index_r0, edge_index_r1, edge_index_r2):
        idx.append(jnp.concatenate([ei[0], pad]).reshape(EROWS, EW))
        idx.append(jnp.concatenate([ei[1], pad]).reshape(EROWS, EW))

    hist = _hist_kernel(idx[0], idx[2], idx[4], idx[1], idx[3], idx[5])
    hist = hist.reshape(2, 6, NPAD)

    x_pad = jnp.zeros((NPAD, F), jnp.float32).at[:N].set(x)
    xn0, xn1, xn2, sin = _tc_pre(x_pad, hist)

    a0, a1, a2 = _segsum_kernel(
        xn0, xn1, xn2, idx[0], idx[1], idx[2], idx[3], idx[4], idx[5])

    W = jnp.stack([W_r0, W_r1, W_r2])
    b = jnp.stack([b_r0, b_r1, b_r2])
    out = _tc_post(a0, a1, a2, sin, W, b, W_att, v_att.reshape(1, ATT))
    return out[:N]


# speedup vs baseline: 6.7463x; 2.4369x over previous
import functools

import jax
import jax.numpy as jnp
from jax import lax
from jax.experimental import pallas as pl
from jax.experimental.pallas import tpu as pltpu
from jax.experimental.pallas import tpu_sc as plsc

N = 10000
F = 128
HID = 256
ATT = 64
E = 320000

NPAD = 10240
EPAD = 327680
EW = 64
EROWS = EPAD // EW
NW = 32
ROWS_PER_W = EPAD // EW // NW
OUTER = 5
INNER = ROWS_PER_W // OUTER
TSLICE = NPAD // 16

SEG_INNER = 16
ROWS_C0 = 2560
ROWS_C1 = EROWS - ROWS_C0
WROWS_C0 = ROWS_C0 // 16
WROWS_C1 = ROWS_C1 // 16
OUTER_C0 = WROWS_C0 // SEG_INNER
OUTER_C1 = WROWS_C1 // SEG_INNER

_mesh = plsc.VectorSubcoreMesh(
    core_axis_name="c", subcore_axis_name="s", num_cores=2, num_subcores=16)


@functools.partial(
    pl.kernel,
    out_type=jax.ShapeDtypeStruct((2, 6 * NPAD), jnp.float32),
    mesh=_mesh,
    scratch_types=[
        [pltpu.VMEM((INNER, EW), jnp.int32) for _ in range(2)],
        pltpu.VMEM((EW,), jnp.float32),
        pltpu.VMEM((TSLICE,), jnp.float32),
        [pltpu.VMEM_SHARED((NPAD,), jnp.float32) for _ in range(6)],
        pltpu.SemaphoreType.DMA,
    ],
)
def _hist_kernel(e0, e1, e2, e3, e4, e5, out, idx_v, ones_v, zero_v, hists,
                 ssem):
    c = lax.axis_index("c")
    s = lax.axis_index("s")
    w = s * 2 + c

    ones16 = jnp.ones((16,), jnp.float32)
    zero16 = jnp.zeros((16,), jnp.float32)

    def init_ones(i, _):
        ones_v[pl.ds(i * 16, 16)] = ones16
        return _

    lax.fori_loop(0, EW // 16, init_ones, None)

    def init_zero(i, _):
        zero_v[pl.ds(i * 16, 16)] = zero16
        return _

    lax.fori_loop(0, TSLICE // 16, init_zero, None)

    for a in range(6):
        pltpu.sync_copy(zero_v, hists[a].at[pl.ds(s * TSLICE, TSLICE)])
    plsc.subcore_barrier()

    wb = w * ROWS_PER_W
    for a, ehbm in enumerate((e0, e1, e2, e3, e4, e5)):
        h_ref = hists[a]

        def outer(cc, _, ehbm=ehbm, h_ref=h_ref):
            pltpu.sync_copy(ehbm.at[pl.ds(wb + cc * INNER, INNER)], idx_v[0])
            sds = [pltpu.async_copy(ones_v, h_ref.at[idx_v[0].at[j]], ssem,
                                    add=True)
                   for j in range(INNER)]
            for sdd in sds:
                sdd.wait()
            return _

        lax.fori_loop(0, OUTER, outer, None)

    plsc.subcore_barrier()
    for a in range(6):
        pltpu.sync_copy(
            hists[a].at[pl.ds(s * TSLICE, TSLICE)],
            out.at[c, pl.ds(a * NPAD + s * TSLICE, TSLICE)])


@functools.partial(
    pl.kernel,
    out_type=[jax.ShapeDtypeStruct((2, NPAD, F), jnp.float32)
              for _ in range(3)],
    mesh=_mesh,
    scratch_types=[
        pltpu.VMEM((SEG_INNER, EW), jnp.int32),
        pltpu.VMEM((SEG_INNER, EW), jnp.int32),
        [pltpu.VMEM((EW, F), jnp.float32) for _ in range(4)],
        pltpu.VMEM((16, F), jnp.float32),
        pltpu.VMEM_SHARED((NPAD, F), jnp.float32),
        [pltpu.SemaphoreType.DMA for _ in range(4)],
        [pltpu.SemaphoreType.DMA for _ in range(4)],
    ],
)
def _segsum_kernel(xn0, xn1, xn2, s0, d0, s1, d1, s2, d2,
                   o0, o1, o2, si_v, di_v, bufs, zero_v, agg, gsems, ssems):
    c = lax.axis_index("c")
    s = lax.axis_index("s")
    nbuf = len(bufs)

    wbase = jnp.where(c == 0, s * WROWS_C0, ROWS_C0 + s * WROWS_C1)
    n_outer = jnp.where(c == 0, OUTER_C0, OUTER_C1)

    z16 = jnp.zeros((16,), jnp.float32)

    def zrow(r, _):
        for q in range(F // 16):
            zero_v[r, pl.ds(q * 16, 16)] = z16
        return _

    lax.fori_loop(0, 16, zrow, None)

    xns = (xn0, xn1, xn2)
    srcs = (s0, s1, s2)
    dsts = (d0, d1, d2)
    outs = (o0, o1, o2)

    for r in range(3):
        def zslice(k, _):
            pltpu.sync_copy(zero_v, agg.at[pl.ds(s * TSLICE + k * 16, 16)])
            return _

        lax.fori_loop(0, TSLICE // 16, zslice, None)
        plsc.subcore_barrier()

        def outer(cc, _, r=r):
            base = wbase + cc * SEG_INNER
            pltpu.sync_copy(srcs[r].at[pl.ds(base, SEG_INNER)], si_v)
            pltpu.sync_copy(dsts[r].at[pl.ds(base, SEG_INNER)], di_v)
            gd = [None] * SEG_INNER
            sd = [None] * SEG_INNER
            for j in range(nbuf):
                b = j % nbuf
                gd[j] = pltpu.async_copy(
                    xns[r].at[si_v.at[j]], bufs[b], gsems[b])
            for j in range(SEG_INNER):
                b = j % nbuf
                gd[j].wait()
                sd[j] = pltpu.async_copy(
                    bufs[b], agg.at[di_v.at[j]], ssems[b], add=True)
                jf = j - 2
                if jf >= 0:
                    sd[jf].wait()
                    nxt = jf + nbuf
                    if nxt < SEG_INNER:
                        nb = nxt % nbuf
                        gd[nxt] = pltpu.async_copy(
                            xns[r].at[si_v.at[nxt]], bufs[nb], gsems[nb])
            sd[SEG_INNER - 2].wait()
            sd[SEG_INNER - 1].wait()
            return _

        lax.fori_loop(0, n_outer, outer, None)
        plsc.subcore_barrier()
        pltpu.sync_copy(agg.at[pl.ds(s * TSLICE, TSLICE)],
                        outs[r].at[c, pl.ds(s * TSLICE, TSLICE)])
        if r < 2:
            plsc.subcore_barrier()


PB = 2048


def _pre_body(x_ref, hist_ref, xn0_ref, xn1_ref, xn2_ref, sin_ref):
    h = hist_ref[...]
    deg = h[0] + h[1]
    sc = lax.rsqrt(jnp.maximum(deg, 1.0))
    xv = x_ref[...]
    xn0_ref[...] = xv * sc[0][:, None]
    xn1_ref[...] = xv * sc[1][:, None]
    xn2_ref[...] = xv * sc[2][:, None]
    sin_ref[...] = sc[3:6]


def _tc_pre(x_pad, hist):
    grid = (NPAD // PB,)
    return pl.pallas_call(
        _pre_body,
        grid=grid,
        in_specs=[
            pl.BlockSpec((PB, F), lambda i: (i, 0)),
            pl.BlockSpec((2, 6, PB), lambda i: (0, 0, i)),
        ],
        out_specs=[
            pl.BlockSpec((PB, F), lambda i: (i, 0)),
            pl.BlockSpec((PB, F), lambda i: (i, 0)),
            pl.BlockSpec((PB, F), lambda i: (i, 0)),
            pl.BlockSpec((3, PB), lambda i: (0, i)),
        ],
        out_shape=[jax.ShapeDtypeStruct((NPAD, F), jnp.float32)] * 3
        + [jax.ShapeDtypeStruct((3, NPAD), jnp.float32)],
    )(x_pad, hist)


PB2 = 1024


def _post_body(a0_ref, a1_ref, a2_ref, sin_ref, w_ref, b_ref, wa_ref, va_ref,
               out_ref):
    sin = sin_ref[...]
    hs = []
    for r, ar in enumerate((a0_ref, a1_ref, a2_ref)):
        av = ar[...]
        asum = (av[0] + av[1]) * sin[r][:, None]
        hs.append(jnp.dot(asum, w_ref[r], preferred_element_type=jnp.float32)
                  + b_ref[r][None, :])
    h = jnp.stack(hs)
    t = jnp.tanh(jnp.dot(h, wa_ref[...],
                         preferred_element_type=jnp.float32))
    e = jnp.sum(t * va_ref[0][None, None, :], axis=-1)
    m = jnp.max(e, axis=0)
    ew = jnp.exp(e - m[None, :])
    alpha = ew / jnp.sum(ew, axis=0)[None, :]
    out_ref[...] = (alpha[0][:, None] * h[0] + alpha[1][:, None] * h[1]
                    + alpha[2][:, None] * h[2])


def _tc_post(a0, a1, a2, sin, W, b, Wa, va):
    grid = (NPAD // PB2,)
    return pl.pallas_call(
        _post_body,
        grid=grid,
        in_specs=[
            pl.BlockSpec((2, PB2, F), lambda i: (0, i, 0)),
            pl.BlockSpec((2, PB2, F), lambda i: (0, i, 0)),
            pl.BlockSpec((2, PB2, F), lambda i: (0, i, 0)),
            pl.BlockSpec((3, PB2), lambda i: (0, i)),
            pl.BlockSpec((3, F, HID), lambda i: (0, 0, 0)),
            pl.BlockSpec((3, HID), lambda i: (0, 0)),
            pl.BlockSpec((HID, ATT), lambda i: (0, 0)),
            pl.BlockSpec((1, ATT), lambda i: (0, 0)),
        ],
        out_specs=pl.BlockSpec((PB2, HID), lambda i: (i, 0)),
        out_shape=jax.ShapeDtypeStruct((NPAD, HID), jnp.float32),
    )(a0, a1, a2, sin, W, b, Wa, va)


def kernel(x, edge_index_r0, edge_index_r1, edge_index_r2,
           W_r0, W_r1, W_r2, b_r0, b_r1, b_r2, W_att, v_att):
    pad = (NPAD - 224) + (jnp.arange(EPAD - E, dtype=jnp.int32) % 224)
    idx = []
    for ei in (edge_index_r0, edge_index_r1, edge_index_r2):
        idx.append(jnp.concatenate([ei[0], pad]).reshape(EROWS, EW))
        idx.append(jnp.concatenate([ei[1], pad]).reshape(EROWS, EW))

    hist = _hist_kernel(idx[0], idx[2], idx[4], idx[1], idx[3], idx[5])
    hist = hist.reshape(2, 6, NPAD)

    x_pad = jnp.zeros((NPAD, F), jnp.float32).at[:N].set(x)
    xn0, xn1, xn2, sin = _tc_pre(x_pad, hist)

    a0, a1, a2 = _segsum_kernel(
        xn0, xn1, xn2, idx[0], idx[1], idx[2], idx[3], idx[4], idx[5])

    W = jnp.stack([W_r0, W_r1, W_r2])
    b = jnp.stack([b_r0, b_r1, b_r2])
    out = _tc_post(a0, a1, a2, sin, W, b, W_att, v_att.reshape(1, ATT))
    return out[:N]


# speedup vs baseline: 7.3555x; 1.0903x over previous
import functools

import jax
import jax.numpy as jnp
from jax import lax
from jax.experimental import pallas as pl
from jax.experimental.pallas import tpu as pltpu
from jax.experimental.pallas import tpu_sc as plsc

N = 10000
F = 128
HID = 256
ATT = 64
E = 320000

NPAD = 10240
EPAD = 327680
EW = 64
EROWS = EPAD // EW
NW = 32
ROWS_PER_W = EPAD // EW // NW
OUTER = 5
INNER = ROWS_PER_W // OUTER
TSLICE = NPAD // 16

SEG_INNER = 16
ROWS_C0 = 2560
ROWS_C1 = EROWS - ROWS_C0
WROWS_C0 = ROWS_C0 // 16
WROWS_C1 = ROWS_C1 // 16
OUTER_C0 = WROWS_C0 // SEG_INNER
OUTER_C1 = WROWS_C1 // SEG_INNER

_mesh = plsc.VectorSubcoreMesh(
    core_axis_name="c", subcore_axis_name="s", num_cores=2, num_subcores=16)


@functools.partial(
    pl.kernel,
    out_type=jax.ShapeDtypeStruct((2, 6 * NPAD), jnp.float32),
    mesh=_mesh,
    scratch_types=[
        [pltpu.VMEM((INNER, EW), jnp.int32) for _ in range(2)],
        pltpu.VMEM((EW,), jnp.float32),
        pltpu.VMEM((TSLICE,), jnp.float32),
        [pltpu.VMEM_SHARED((NPAD,), jnp.float32) for _ in range(6)],
        pltpu.SemaphoreType.DMA,
    ],
)
def _hist_kernel(e0, e1, e2, e3, e4, e5, out, idx_v, ones_v, zero_v, hists,
                 ssem):
    c = lax.axis_index("c")
    s = lax.axis_index("s")
    w = s * 2 + c

    ones16 = jnp.ones((16,), jnp.float32)
    zero16 = jnp.zeros((16,), jnp.float32)

    def init_ones(i, _):
        ones_v[pl.ds(i * 16, 16)] = ones16
        return _

    lax.fori_loop(0, EW // 16, init_ones, None)

    def init_zero(i, _):
        zero_v[pl.ds(i * 16, 16)] = zero16
        return _

    lax.fori_loop(0, TSLICE // 16, init_zero, None)

    for a in range(6):
        pltpu.sync_copy(zero_v, hists[a].at[pl.ds(s * TSLICE, TSLICE)])
    plsc.subcore_barrier()

    wb = w * ROWS_PER_W
    for a, ehbm in enumerate((e0, e1, e2, e3, e4, e5)):
        h_ref = hists[a]

        def outer(cc, _, ehbm=ehbm, h_ref=h_ref):
            pltpu.sync_copy(ehbm.at[pl.ds(wb + cc * INNER, INNER)], idx_v[0])
            sds = [pltpu.async_copy(ones_v, h_ref.at[idx_v[0].at[j]], ssem,
                                    add=True)
                   for j in range(INNER)]
            for sdd in sds:
                sdd.wait()
            return _

        lax.fori_loop(0, OUTER, outer, None)

    plsc.subcore_barrier()
    for a in range(6):
        pltpu.sync_copy(
            hists[a].at[pl.ds(s * TSLICE, TSLICE)],
            out.at[c, pl.ds(a * NPAD + s * TSLICE, TSLICE)])


@functools.partial(
    pl.kernel,
    out_type=[jax.ShapeDtypeStruct((2, NPAD, F), jnp.float32)
              for _ in range(3)],
    mesh=_mesh,
    scratch_types=[
        pltpu.VMEM((SEG_INNER, EW), jnp.int32),
        pltpu.VMEM((SEG_INNER, EW), jnp.int32),
        [pltpu.VMEM((EW, F), jnp.float32) for _ in range(5)],
        pltpu.VMEM((16, F), jnp.float32),
        pltpu.VMEM_SHARED((NPAD, F), jnp.float32),
        [pltpu.SemaphoreType.DMA for _ in range(5)],
        [pltpu.SemaphoreType.DMA for _ in range(5)],
    ],
)
def _segsum_kernel(xn0, xn1, xn2, s0, d0, s1, d1, s2, d2,
                   o0, o1, o2, si_v, di_v, bufs, zero_v, agg, gsems, ssems):
    c = lax.axis_index("c")
    s = lax.axis_index("s")
    nbuf = len(bufs)

    wbase = jnp.where(c == 0, s * WROWS_C0, ROWS_C0 + s * WROWS_C1)
    n_outer = jnp.where(c == 0, OUTER_C0, OUTER_C1)

    z16 = jnp.zeros((16,), jnp.float32)

    def zrow(r, _):
        for q in range(F // 16):
            zero_v[r, pl.ds(q * 16, 16)] = z16
        return _

    lax.fori_loop(0, 16, zrow, None)

    xns = (xn0, xn1, xn2)
    srcs = (s0, s1, s2)
    dsts = (d0, d1, d2)
    outs = (o0, o1, o2)

    for r in range(3):
        def zslice(k, _):
            pltpu.sync_copy(zero_v, agg.at[pl.ds(s * TSLICE + k * 16, 16)])
            return _

        lax.fori_loop(0, TSLICE // 16, zslice, None)
        plsc.subcore_barrier()

        def outer(cc, _, r=r):
            base = wbase + cc * SEG_INNER
            pltpu.sync_copy(srcs[r].at[pl.ds(base, SEG_INNER)], si_v)
            pltpu.sync_copy(dsts[r].at[pl.ds(base, SEG_INNER)], di_v)
            gd = [None] * SEG_INNER
            sd = [None] * SEG_INNER
            for j in range(nbuf):
                b = j % nbuf
                gd[j] = pltpu.async_copy(
                    xns[r].at[si_v.at[j]], bufs[b], gsems[b])
            for j in range(SEG_INNER):
                b = j % nbuf
                gd[j].wait()
                sd[j] = pltpu.async_copy(
                    bufs[b], agg.at[di_v.at[j]], ssems[b], add=True)
                jf = j - 2
                if jf >= 0:
                    sd[jf].wait()
                    nxt = jf + nbuf
                    if nxt < SEG_INNER:
                        nb = nxt % nbuf
                        gd[nxt] = pltpu.async_copy(
                            xns[r].at[si_v.at[nxt]], bufs[nb], gsems[nb])
            sd[SEG_INNER - 2].wait()
            sd[SEG_INNER - 1].wait()
            return _

        lax.fori_loop(0, n_outer, outer, None)
        plsc.subcore_barrier()
        pltpu.sync_copy(agg.at[pl.ds(s * TSLICE, TSLICE)],
                        outs[r].at[c, pl.ds(s * TSLICE, TSLICE)])
        if r < 2:
            plsc.subcore_barrier()


PB = 2048


def _pre_body(x_ref, hist_ref, xn0_ref, xn1_ref, xn2_ref, sin_ref):
    h = hist_ref[...]
    deg = h[0] + h[1]
    sc = lax.rsqrt(jnp.maximum(deg, 1.0))
    xv = x_ref[...]
    xn0_ref[...] = xv * sc[0][:, None]
    xn1_ref[...] = xv * sc[1][:, None]
    xn2_ref[...] = xv * sc[2][:, None]
    sin_ref[...] = sc[3:6]


def _tc_pre(x_pad, hist):
    grid = (NPAD // PB,)
    return pl.pallas_call(
        _pre_body,
        grid=grid,
        in_specs=[
            pl.BlockSpec((PB, F), lambda i: (i, 0)),
            pl.BlockSpec((2, 6, PB), lambda i: (0, 0, i)),
        ],
        out_specs=[
            pl.BlockSpec((PB, F), lambda i: (i, 0)),
            pl.BlockSpec((PB, F), lambda i: (i, 0)),
            pl.BlockSpec((PB, F), lambda i: (i, 0)),
            pl.BlockSpec((3, PB), lambda i: (0, i)),
        ],
        out_shape=[jax.ShapeDtypeStruct((NPAD, F), jnp.float32)] * 3
        + [jax.ShapeDtypeStruct((3, NPAD), jnp.float32)],
    )(x_pad, hist)


PB2 = 1000


def _post_body(a0_ref, a1_ref, a2_ref, sin_ref, w_ref, b_ref, wa_ref, va_ref,
               out_ref):
    sin = sin_ref[...]
    hs = []
    for r, ar in enumerate((a0_ref, a1_ref, a2_ref)):
        av = ar[...]
        asum = (av[0] + av[1]) * sin[r]
        hs.append(jnp.dot(asum, w_ref[r], preferred_element_type=jnp.float32)
                  + b_ref[r][None, :])
    h = jnp.stack(hs)
    t = jnp.tanh(jnp.dot(h, wa_ref[...],
                         preferred_element_type=jnp.float32))
    e = jnp.sum(t * va_ref[0][None, None, :], axis=-1)
    m = jnp.max(e, axis=0)
    ew = jnp.exp(e - m[None, :])
    alpha = ew / jnp.sum(ew, axis=0)[None, :]
    out_ref[...] = (alpha[0][:, None] * h[0] + alpha[1][:, None] * h[1]
                    + alpha[2][:, None] * h[2])


def _tc_post(a0, a1, a2, sin, W, b, Wa, va):
    grid = (N // PB2,)
    return pl.pallas_call(
        _post_body,
        grid=grid,
        in_specs=[
            pl.BlockSpec((2, PB2, F), lambda i: (0, i, 0)),
            pl.BlockSpec((2, PB2, F), lambda i: (0, i, 0)),
            pl.BlockSpec((2, PB2, F), lambda i: (0, i, 0)),
            pl.BlockSpec((3, PB2, 1), lambda i: (0, i, 0)),
            pl.BlockSpec((3, F, HID), lambda i: (0, 0, 0)),
            pl.BlockSpec((3, HID), lambda i: (0, 0)),
            pl.BlockSpec((HID, ATT), lambda i: (0, 0)),
            pl.BlockSpec((1, ATT), lambda i: (0, 0)),
        ],
        out_specs=pl.BlockSpec((PB2, HID), lambda i: (i, 0)),
        out_shape=jax.ShapeDtypeStruct((N, HID), jnp.float32),
    )(a0, a1, a2, sin.reshape(3, NPAD, 1), W, b, Wa, va)


def kernel(x, edge_index_r0, edge_index_r1, edge_index_r2,
           W_r0, W_r1, W_r2, b_r0, b_r1, b_r2, W_att, v_att):
    pad = (NPAD - 224) + (jnp.arange(EPAD - E, dtype=jnp.int32) % 224)
    idx = []
    for ei in (edge_index_r0, edge_index_r1, edge_index_r2):
        idx.append(jnp.concatenate([ei[0], pad]).reshape(EROWS, EW))
        idx.append(jnp.concatenate([ei[1], pad]).reshape(EROWS, EW))

    hist = _hist_kernel(idx[0], idx[2], idx[4], idx[1], idx[3], idx[5])
    hist = hist.reshape(2, 6, NPAD)

    x_pad = jnp.zeros((NPAD, F), jnp.float32).at[:N].set(x)
    xn0, xn1, xn2, sin = _tc_pre(x_pad, hist)

    a0, a1, a2 = _segsum_kernel(
        xn0, xn1, xn2, idx[0], idx[1], idx[2], idx[3], idx[4], idx[5])

    W = jnp.stack([W_r0, W_r1, W_r2])
    b = jnp.stack([b_r0, b_r1, b_r2])
    return _tc_post(a0, a1, a2, sin, W, b, W_att, v_att.reshape(1, ATT))


# speedup vs baseline: 7.9828x; 1.0853x over previous
import functools

import jax
import jax.numpy as jnp
from jax import lax
from jax.experimental import pallas as pl
from jax.experimental.pallas import tpu as pltpu
from jax.experimental.pallas import tpu_sc as plsc

N = 10000
F = 128
HID = 256
ATT = 64
E = 320000

NPAD = 10240
EW = 50
EROWS = E // EW
NW = 32
WROWS = EROWS // NW
SEG_INNER = 40
OUTER = WROWS // SEG_INNER
NBUF = 5
TSLICE = NPAD // 16

_mesh = plsc.VectorSubcoreMesh(
    core_axis_name="c", subcore_axis_name="s", num_cores=2, num_subcores=16)


@functools.partial(
    pl.kernel,
    out_type=jax.ShapeDtypeStruct((2, 6 * NPAD), jnp.float32),
    mesh=_mesh,
    scratch_types=[
        pltpu.VMEM((SEG_INNER, EW), jnp.int32),
        pltpu.VMEM((64,), jnp.float32),
        pltpu.VMEM((TSLICE,), jnp.float32),
        [pltpu.VMEM_SHARED((NPAD,), jnp.float32) for _ in range(6)],
        pltpu.SemaphoreType.DMA,
    ],
)
def _hist_kernel(e0, e1, e2, out, idx_v, ones_v, zero_v, hists, ssem):
    c = lax.axis_index("c")
    s = lax.axis_index("s")
    w = s * 2 + c

    ones16 = jnp.ones((16,), jnp.float32)
    zero16 = jnp.zeros((16,), jnp.float32)

    def init_ones(i, _):
        ones_v[pl.ds(i * 16, 16)] = ones16
        return _

    lax.fori_loop(0, 4, init_ones, None)

    def init_zero(i, _):
        zero_v[pl.ds(i * 16, 16)] = zero16
        return _

    lax.fori_loop(0, TSLICE // 16, init_zero, None)

    for a in range(6):
        pltpu.sync_copy(zero_v, hists[a].at[pl.ds(s * TSLICE, TSLICE)])
    plsc.subcore_barrier()

    wb = w * WROWS
    ones_row = ones_v.at[pl.ds(0, EW)]
    for r, ehbm in enumerate((e0, e1, e2)):
        for half in range(2):
            h_ref = hists[half * 3 + r]

            def outer(cc, _, ehbm=ehbm, h_ref=h_ref, half=half):
                pltpu.sync_copy(
                    ehbm.at[half, pl.ds(wb + cc * SEG_INNER, SEG_INNER)],
                    idx_v)
                sds = [pltpu.async_copy(ones_row, h_ref.at[idx_v.at[j]],
                                        ssem, add=True)
                       for j in range(SEG_INNER)]
                for sdd in sds:
                    sdd.wait()
                return _

            lax.fori_loop(0, OUTER, outer, None)

    plsc.subcore_barrier()
    for a in range(6):
        pltpu.sync_copy(
            hists[a].at[pl.ds(s * TSLICE, TSLICE)],
            out.at[c, pl.ds(a * NPAD + s * TSLICE, TSLICE)])


@functools.partial(
    pl.kernel,
    out_type=[jax.ShapeDtypeStruct((2, NPAD, F), jnp.float32)
              for _ in range(3)],
    mesh=_mesh,
    scratch_types=[
        pltpu.VMEM((SEG_INNER, EW), jnp.int32),
        pltpu.VMEM((SEG_INNER, EW), jnp.int32),
        [pltpu.VMEM((EW, F), jnp.float32) for _ in range(NBUF)],
        pltpu.VMEM((16, F), jnp.float32),
        pltpu.VMEM_SHARED((NPAD, F), jnp.float32),
        [pltpu.SemaphoreType.DMA for _ in range(NBUF)],
        [pltpu.SemaphoreType.DMA for _ in range(NBUF)],
    ],
)
def _segsum_kernel(xn0, xn1, xn2, e0, e1, e2,
                   o0, o1, o2, si_v, di_v, bufs, zero_v, agg, gsems, ssems):
    c = lax.axis_index("c")
    s = lax.axis_index("s")
    w = s * 2 + c
    wbase = w * WROWS

    z16 = jnp.zeros((16,), jnp.float32)

    def zrow(r, _):
        for q in range(F // 16):
            zero_v[r, pl.ds(q * 16, 16)] = z16
        return _

    lax.fori_loop(0, 16, zrow, None)

    xns = (xn0, xn1, xn2)
    edges = (e0, e1, e2)
    outs = (o0, o1, o2)

    for r in range(3):
        def zslice(k, _):
            pltpu.sync_copy(zero_v, agg.at[pl.ds(s * TSLICE + k * 16, 16)])
            return _

        lax.fori_loop(0, TSLICE // 16, zslice, None)
        plsc.subcore_barrier()

        def outer(cc, _, r=r):
            base = wbase + cc * SEG_INNER
            pltpu.sync_copy(edges[r].at[0, pl.ds(base, SEG_INNER)], si_v)
            pltpu.sync_copy(edges[r].at[1, pl.ds(base, SEG_INNER)], di_v)
            gd = [None] * SEG_INNER
            sd = [None] * SEG_INNER
            for j in range(NBUF):
                gd[j] = pltpu.async_copy(
                    xns[r].at[si_v.at[j]], bufs[j], gsems[j])
            for j in range(SEG_INNER):
                b = j % NBUF
                gd[j].wait()
                sd[j] = pltpu.async_copy(
                    bufs[b], agg.at[di_v.at[j]], ssems[b], add=True)
                jf = j - 2
                if jf >= 0:
                    sd[jf].wait()
                    nxt = jf + NBUF
                    if nxt < SEG_INNER:
                        nb = nxt % NBUF
                        gd[nxt] = pltpu.async_copy(
                            xns[r].at[si_v.at[nxt]], bufs[nb], gsems[nb])
            sd[SEG_INNER - 2].wait()
            sd[SEG_INNER - 1].wait()
            return _

        lax.fori_loop(0, OUTER, outer, None)
        plsc.subcore_barrier()
        pltpu.sync_copy(agg.at[pl.ds(s * TSLICE, TSLICE)],
                        outs[r].at[c, pl.ds(s * TSLICE, TSLICE)])
        if r < 2:
            plsc.subcore_barrier()


PB = 2048


def _pre_body(x_ref, hist_ref, xn0_ref, xn1_ref, xn2_ref, sin_ref):
    h = hist_ref[...]
    deg = h[0] + h[1]
    sc = lax.rsqrt(jnp.maximum(deg, 1.0))
    xv = x_ref[...]
    xn0_ref[...] = xv * sc[0][:, None]
    xn1_ref[...] = xv * sc[1][:, None]
    xn2_ref[...] = xv * sc[2][:, None]
    sin_ref[...] = sc[3:6]


def _tc_pre(x, hist):
    grid = (NPAD // PB,)
    return pl.pallas_call(
        _pre_body,
        grid=grid,
        in_specs=[
            pl.BlockSpec((PB, F), lambda i: (i, 0)),
            pl.BlockSpec((2, 6, PB), lambda i: (0, 0, i)),
        ],
        out_specs=[
            pl.BlockSpec((PB, F), lambda i: (i, 0)),
            pl.BlockSpec((PB, F), lambda i: (i, 0)),
            pl.BlockSpec((PB, F), lambda i: (i, 0)),
            pl.BlockSpec((3, PB), lambda i: (0, i)),
        ],
        out_shape=[jax.ShapeDtypeStruct((N, F), jnp.float32)] * 3
        + [jax.ShapeDtypeStruct((3, NPAD), jnp.float32)],
    )(x, hist)


PB2 = 1000


def _post_body(a0_ref, a1_ref, a2_ref, sin_ref, w_ref, b_ref, wa_ref, va_ref,
               out_ref):
    sin = sin_ref[...]
    hs = []
    for r, ar in enumerate((a0_ref, a1_ref, a2_ref)):
        av = ar[...]
        asum = (av[0] + av[1]) * sin[r]
        hs.append(jnp.dot(asum, w_ref[r], preferred_element_type=jnp.float32)
                  + b_ref[r][None, :])
    h = jnp.stack(hs)
    t = jnp.tanh(jnp.dot(h, wa_ref[...],
                         preferred_element_type=jnp.float32))
    e = jnp.sum(t * va_ref[0][None, None, :], axis=-1)
    m = jnp.max(e, axis=0)
    ew = jnp.exp(e - m[None, :])
    alpha = ew / jnp.sum(ew, axis=0)[None, :]
    out_ref[...] = (alpha[0][:, None] * h[0] + alpha[1][:, None] * h[1]
                    + alpha[2][:, None] * h[2])


def _tc_post(a0, a1, a2, sin, W, b, Wa, va):
    grid = (N // PB2,)
    return pl.pallas_call(
        _post_body,
        grid=grid,
        in_specs=[
            pl.BlockSpec((2, PB2, F), lambda i: (0, i, 0)),
            pl.BlockSpec((2, PB2, F), lambda i: (0, i, 0)),
            pl.BlockSpec((2, PB2, F), lambda i: (0, i, 0)),
            pl.BlockSpec((3, PB2, 1), lambda i: (0, i, 0)),
            pl.BlockSpec((3, F, HID), lambda i: (0, 0, 0)),
            pl.BlockSpec((3, HID), lambda i: (0, 0)),
            pl.BlockSpec((HID, ATT), lambda i: (0, 0)),
            pl.BlockSpec((1, ATT), lambda i: (0, 0)),
        ],
        out_specs=pl.BlockSpec((PB2, HID), lambda i: (i, 0)),
        out_shape=jax.ShapeDtypeStruct((N, HID), jnp.float32),
    )(a0, a1, a2, sin.reshape(3, NPAD, 1), W, b, Wa, va)


def kernel(x, edge_index_r0, edge_index_r1, edge_index_r2,
           W_r0, W_r1, W_r2, b_r0, b_r1, b_r2, W_att, v_att):
    e0 = edge_index_r0.reshape(2, EROWS, EW)
    e1 = edge_index_r1.reshape(2, EROWS, EW)
    e2 = edge_index_r2.reshape(2, EROWS, EW)

    hist = _hist_kernel(e0, e1, e2)
    hist = hist.reshape(2, 6, NPAD)

    xn0, xn1, xn2, sin = _tc_pre(x, hist)

    a0, a1, a2 = _segsum_kernel(xn0, xn1, xn2, e0, e1, e2)

    W = jnp.stack([W_r0, W_r1, W_r2])
    b = jnp.stack([b_r0, b_r1, b_r2])
    return _tc_post(a0, a1, a2, sin, W, b, W_att, v_att.reshape(1, ATT))
